# baseline (speedup 1.0000x reference)
import functools

import jax
import jax.numpy as jnp
from jax import lax
from jax.experimental import pallas as pl
from jax.experimental.pallas import tpu as pltpu
from jax.experimental.pallas import tpu_sc as plsc

_VOCAB = 1000
_D = 1000
_B = 1024
_H = 20
_NSEG = 5
_SEG = _D // _NSEG
_NC = 2
_NS = 16
_NW = _NC * _NS
_NBLK = _H * (_B // 128)
_BPW = _NBLK // _NW
_NU = _BPW * _NSEG
_TLP = _SEG // 8
_STG = _TLP * 1024


def _body(tab_hbm, idx_hbm, out_hbm, idx_v, in0, in1, st0, st1,
          sg0, sg1, sw0, sw1):
    w = lax.axis_index("s") * _NC + lax.axis_index("c")
    pltpu.sync_copy(idx_hbm.at[pl.ds(w * (_NU * 128), _NU * 128)], idx_v)

    inb = (in0, in1)
    stg = (st0, st1)
    sg = (sg0, sg1)
    sw = (sw0, sw1)

    iota = jnp.arange(16, dtype=jnp.int32)
    rowv = [iota + c0 for c0 in range(0, 128, 16)]

    def gather_args(u, p):
        return (tab_hbm.at[idx_v.at[pl.ds(u * 128, 128)]], inb[p], sg[p])

    def unit_base(u):
        bid = w * _BPW + u // _NSEG
        k = u % _NSEG
        h = bid // 8
        tc = bid % 8
        return ((h * 125 + k * _TLP) * 8 + tc) * 1024

    def transpose(p):
        @plsc.parallel_loop(0, _SEG, step=1, unroll=40)
        def tbody(vv):
            col = jnp.zeros((16,), jnp.int32) + vv
            pos = (vv // 8) * 1024 + (vv % 8) * 128
            for j in range(8):
                val = plsc.load_gather(inb[p], [rowv[j], col])
                stg[p][pl.ds(pos + j * 16, 16)] = val

    def write_unit(u, p):
        base = unit_base(u)
        @plsc.parallel_loop(0, _TLP, step=1, unroll=5)
        def wbody(tl):
            pltpu.async_copy(stg[p].at[pl.ds(tl * 1024, 1024)],
                             out_hbm.at[pl.ds(base + tl * 8192, 1024)],
                             sw[p])

    def drain_writes(p):
        pltpu.make_async_copy(out_hbm.at[pl.ds(0, _STG)], stg[p], sw[p]).wait()

    pltpu.async_copy(*gather_args(0, 0))

    def step(j, c):
        for p in range(2):
            u = j * 2 + p
            @pl.when(u < _NU - 1)
            def _():
                pltpu.async_copy(*gather_args(u + 1, 1 - p))
            pltpu.make_async_copy(*gather_args(u, p)).wait()
            @pl.when(j > 0)
            def _():
                drain_writes(p)
            transpose(p)
            write_unit(u, p)
        return c

    lax.fori_loop(0, (_NU - 1) // 2, step, 0, unroll=False)

    u = _NU - 1
    pltpu.make_async_copy(*gather_args(u, 0)).wait()
    drain_writes(0)
    transpose(0)
    write_unit(u, 0)
    drain_writes(1)
    drain_writes(0)


@functools.partial(
    pl.kernel,
    mesh=plsc.VectorSubcoreMesh(core_axis_name="c", subcore_axis_name="s"),
    out_type=jax.ShapeDtypeStruct((_B * _H * _D,), jnp.float32),
    compiler_params=pltpu.CompilerParams(use_tc_tiling_on_sc=False,
                                        needs_layout_passes=False),
    scratch_types=[
        pltpu.VMEM((_NU * 128,), jnp.int32),
        pltpu.VMEM((128, _SEG), jnp.float32),
        pltpu.VMEM((128, _SEG), jnp.float32),
        pltpu.VMEM((_STG,), jnp.float32),
        pltpu.VMEM((_STG,), jnp.float32),
        pltpu.SemaphoreType.DMA,
        pltpu.SemaphoreType.DMA,
        pltpu.SemaphoreType.DMA,
        pltpu.SemaphoreType.DMA,
    ],
)
def _sc_gather(tab_hbm, idx_hbm, out_hbm, idx_v, in0, in1, st0, st1,
               sg0, sg1, sw0, sw1):
    _body(tab_hbm, idx_hbm, out_hbm, idx_v, in0, in1, st0, st1,
          sg0, sg1, sw0, sw1)


def kernel(x, logits):
    xi = x.astype(jnp.int32)
    a = (xi.T * _NSEG).reshape(_H, 8, 128)
    idx = a[:, :, None, :] + jnp.arange(_NSEG, dtype=jnp.int32)[None, None, :, None]
    idx = idx.reshape(-1)
    tab5 = logits.reshape(_VOCAB * _NSEG, _SEG)
    o1 = _sc_gather(tab5, idx)
    o5 = o1.reshape(_H, 125, 8, 8, 128)
    return o5.transpose(2, 4, 0, 1, 3).reshape(_B, _H, _D)


# speedup vs baseline: 1.0734x; 1.0734x over previous
import functools

import jax
import jax.numpy as jnp
from jax import lax
from jax.experimental import pallas as pl
from jax.experimental.pallas import tpu as pltpu
from jax.experimental.pallas import tpu_sc as plsc

_VOCAB = 1000
_D = 1000
_B = 1024
_H = 20
_NSEG = 5
_SEG = _D // _NSEG
_NC = 2
_NS = 16
_NW = _NC * _NS
_NBLK = _H * (_B // 128)
_BPW = _NBLK // _NW
_NU = _BPW * _NSEG
_TLP = _SEG // 8
_STG = _TLP * 1024


def _body(tab_hbm, xv_hbm, out_hbm, xv, ib0, ib1, ib2, id0, id1, id2,
          st0, st1, sg0, sg1, sg2, sw0, sw1):
    w = lax.axis_index("s") * _NC + lax.axis_index("c")
    pltpu.sync_copy(xv_hbm.at[pl.ds(w * (_BPW * 128), _BPW * 128)], xv)

    inb = (ib0, ib1, ib2)
    idxb = (id0, id1, id2)
    stg = (st0, st1)
    sg = (sg0, sg1, sg2)
    sw = (sw0, sw1)

    iota = jnp.arange(16, dtype=jnp.int32)
    rowv = [iota + c0 for c0 in range(0, 128, 16)]

    def fire_gather(u, g):
        bidl = u // _NSEG
        k = u % _NSEG
        for j in range(8):
            xs = xv[pl.ds(bidl * 128 + j * 16, 16)]
            idxb[g][pl.ds(j * 16, 16)] = xs * _NSEG + k
        pltpu.async_copy(tab_hbm.at[idxb[g]], inb[g], sg[g])

    def wait_gather(g):
        pltpu.make_async_copy(tab_hbm.at[idxb[g]], inb[g], sg[g]).wait()

    def unit_base(u):
        bid = w * _BPW + u // _NSEG
        k = u % _NSEG
        h = bid // 8
        tc = bid % 8
        return ((h * 125 + k * _TLP) * 8 + tc) * 1024

    def transpose(g, p):
        @plsc.parallel_loop(0, _SEG, step=1, unroll=20)
        def tbody(vv):
            col = jnp.zeros((16,), jnp.int32) + vv
            pos = (vv // 8) * 1024 + (vv % 8) * 128
            for j in range(8):
                val = plsc.load_gather(inb[g], [rowv[j], col])
                stg[p][pl.ds(pos + j * 16, 16)] = val

    def write_unit(u, p):
        base = unit_base(u)
        def wbody(tl, c):
            pltpu.async_copy(stg[p].at[pl.ds(tl * 1024, 1024)],
                             out_hbm.at[pl.ds(base + tl * 8192, 1024)],
                             sw[p])
            return c
        lax.fori_loop(0, _TLP, wbody, 0, unroll=False)

    def drain_writes(p):
        pltpu.make_async_copy(out_hbm.at[pl.ds(0, _STG)], stg[p], sw[p]).wait()

    for u0 in range(3):
        fire_gather(u0, u0)

    def step(j, c):
        for t in range(6):
            u = j * 6 + t
            g = t % 3
            p = t % 2
            wait_gather(g)
            if t < 2:
                @pl.when(j > 0)
                def _():
                    drain_writes(p)
            else:
                drain_writes(p)
            transpose(g, p)
            @pl.when(u + 3 < _NU)
            def _():
                fire_gather(u + 3, g)
            write_unit(u, p)
        return c

    lax.fori_loop(0, (_NU - 1) // 6, step, 0, unroll=False)

    wait_gather(0)
    drain_writes(0)
    transpose(0, 0)
    write_unit(_NU - 1, 0)
    drain_writes(1)
    drain_writes(0)


@functools.partial(
    pl.kernel,
    mesh=plsc.VectorSubcoreMesh(core_axis_name="c", subcore_axis_name="s"),
    out_type=jax.ShapeDtypeStruct((_B * _H * _D,), jnp.float32),
    compiler_params=pltpu.CompilerParams(use_tc_tiling_on_sc=False,
                                        needs_layout_passes=False),
    scratch_types=[
        pltpu.VMEM((_BPW * 128,), jnp.int32),
        pltpu.VMEM((128, _SEG), jnp.float32),
        pltpu.VMEM((128, _SEG), jnp.float32),
        pltpu.VMEM((128, _SEG), jnp.float32),
        pltpu.VMEM((128,), jnp.int32),
        pltpu.VMEM((128,), jnp.int32),
        pltpu.VMEM((128,), jnp.int32),
        pltpu.VMEM((_STG,), jnp.float32),
        pltpu.VMEM((_STG,), jnp.float32),
        pltpu.SemaphoreType.DMA,
        pltpu.SemaphoreType.DMA,
        pltpu.SemaphoreType.DMA,
        pltpu.SemaphoreType.DMA,
        pltpu.SemaphoreType.DMA,
    ],
)
def _sc_gather(tab_hbm, xv_hbm, out_hbm, xv, ib0, ib1, ib2, id0, id1, id2,
               st0, st1, sg0, sg1, sg2, sw0, sw1):
    _body(tab_hbm, xv_hbm, out_hbm, xv, ib0, ib1, ib2, id0, id1, id2,
          st0, st1, sg0, sg1, sg2, sw0, sw1)


def kernel(x, logits):
    xi = x.astype(jnp.int32)
    xarr = xi.T.reshape(_H, 8, 128).reshape(-1)
    tab5 = logits.reshape(_VOCAB * _NSEG, _SEG)
    o1 = _sc_gather(tab5, xarr)
    o5 = o1.reshape(_H, 125, 8, 8, 128)
    return o5.transpose(2, 4, 0, 1, 3).reshape(_B, _H, _D)
